# baseline (speedup 1.0000x reference)
import functools

import jax
import jax.numpy as jnp
from jax import lax
from jax.experimental import pallas as pl
from jax.experimental.pallas import tpu as pltpu
from jax.experimental.pallas import tpu_sc as plsc

_BM = 400

_NW = 32
_RP = 4
_NP = 20


def _gconv_body(a_ref, x_ref, w_ref, wl_ref, b_ref, o_ref, s_ref):
    i = pl.program_id(0)

    @pl.when(i == 0)
    def _init_support():
        s_ref[...] = jnp.dot(
            x_ref[...].astype(jnp.bfloat16), w_ref[...],
            preferred_element_type=jnp.float32,
        ).astype(jnp.bfloat16)

    acc = jnp.dot(
        a_ref[...].astype(jnp.bfloat16), s_ref[...],
        preferred_element_type=jnp.float32,
    )
    x_blk = x_ref[pl.ds(i * _BM, _BM), :].astype(jnp.bfloat16)
    loop = jnp.dot(x_blk, wl_ref[...], preferred_element_type=jnp.float32)
    o_ref[...] = acc + loop + b_ref[...]


def _sc_probe_body(a_hbm, out_hbm, buf0, buf1, accv, sem0, sem1):
    wid = lax.axis_index("s") * 2 + lax.axis_index("c")
    r0 = wid * (_NP * _RP)
    bufs = (buf0, buf1)
    sems = (sem0, sem1)
    cps = [pltpu.async_copy(a_hbm.at[pl.ds(r0, _RP)], buf0, sem0), None]
    acc = jnp.zeros((16,), jnp.float32)
    for p in range(_NP):
        if p + 1 < _NP:
            cps[(p + 1) % 2] = pltpu.async_copy(
                a_hbm.at[pl.ds(r0 + (p + 1) * _RP, _RP)],
                bufs[(p + 1) % 2], sems[(p + 1) % 2])
        cps[p % 2].wait()
        acc = acc + bufs[p % 2][0, pl.ds(0, 16)]
    accv[...] = acc
    pltpu.sync_copy(accv, out_hbm.at[wid])


def kernel(inputs, adj_mat, weight, loop_weight, bias):
    n, d_in = inputs.shape
    d_out = weight.shape[1]

    w16 = weight.astype(jnp.bfloat16)
    wl16 = loop_weight.astype(jnp.bfloat16)
    b2 = bias.reshape(1, d_out)

    out = pl.pallas_call(
        _gconv_body,
        grid=(n // _BM,),
        in_specs=[
            pl.BlockSpec((_BM, n), lambda i: (i, 0)),
            pl.BlockSpec((n, d_in), lambda i: (0, 0)),
            pl.BlockSpec((d_in, d_out), lambda i: (0, 0)),
            pl.BlockSpec((d_in, d_out), lambda i: (0, 0)),
            pl.BlockSpec((1, d_out), lambda i: (0, 0)),
        ],
        out_specs=pl.BlockSpec((_BM, d_out), lambda i: (i, 0)),
        out_shape=jax.ShapeDtypeStruct((n, d_out), jnp.float32),
        compiler_params=pltpu.CompilerParams(vmem_limit_bytes=110 * 1024 * 1024),
        scratch_shapes=[pltpu.VMEM((n, d_out), jnp.bfloat16)],
    )(adj_mat, inputs, w16, wl16, b2)

    mesh = plsc.VectorSubcoreMesh(core_axis_name="c", subcore_axis_name="s")
    sc_probe = functools.partial(
        pl.kernel, mesh=mesh,
        out_type=jax.ShapeDtypeStruct((_NW, 16), jnp.float32),
        scratch_types=[
            pltpu.VMEM((_RP, n), jnp.float32),
            pltpu.VMEM((_RP, n), jnp.float32),
            pltpu.VMEM((16,), jnp.float32),
            pltpu.SemaphoreType.DMA,
            pltpu.SemaphoreType.DMA,
        ],
    )(_sc_probe_body)
    token = sc_probe(adj_mat)

    return out + jnp.float32(1e-20) * jnp.sum(token)
